# baseline (speedup 1.0000x reference)
import functools

import jax
import jax.numpy as jnp
from jax import lax
from jax.experimental import pallas as pl
from jax.experimental.pallas import tpu as pltpu

_B, _C, _N, _K, _OUT = 16, 64, 1024, 20, 64
_RB = 256


def _edgeconv_body(xb_ref, xr_ref, ones_ref, w1d_ref, w1b_ref, b1_ref, w2_ref,
                   b2_ref, out_ref, cur_ref):
    xb = xb_ref[0]
    xr = xr_ref[0]
    sqb = lax.dot_general(xb * xb, ones_ref[...], (((1,), (0,)), ((), ())),
                          preferred_element_type=jnp.float32)
    dot = lax.dot_general(xb, xr, (((1,), (1,)), ((), ())),
                          preferred_element_type=jnp.float32)
    cur_ref[...] = sqb - 2.0 * dot

    v = jnp.dot(xb, w1b_ref[...], preferred_element_type=jnp.float32)
    u = jnp.dot(xr, w1d_ref[...], preferred_element_type=jnp.float32)
    ub = u + b1_ref[...][None, :]
    w2 = w2_ref[...]
    b2 = b2_ref[...][None, :]

    iota = lax.broadcasted_iota(jnp.int32, (_N, _RB), 0)

    def body(_, acc):
        cur = cur_ref[...]
        m = jnp.min(cur, axis=0, keepdims=True)
        ismin = cur == m
        amin = jnp.min(jnp.where(ismin, iota, _N), axis=0, keepdims=True)
        onehot = iota == amin
        vj = lax.dot_general(onehot.astype(jnp.float32), v,
                             (((0,), (0,)), ((), ())),
                             preferred_element_type=jnp.float32)
        e = jnp.maximum(ub + vj, 0.0)
        o = jnp.maximum(jnp.dot(e, w2, preferred_element_type=jnp.float32) + b2,
                        0.0)
        cur_ref[...] = jnp.where(onehot, jnp.inf, cur)
        return jnp.maximum(acc, o)

    acc = lax.fori_loop(0, _K, body,
                        jnp.full((_RB, _OUT), -jnp.inf, jnp.float32))
    out_ref[0] = acc


@functools.partial(jax.jit, static_argnames=("interpret",))
def kernel(x, W1, b1, W2, b2, interpret=False):
    xf = jnp.transpose(x[..., 0], (0, 2, 1))
    w1a, w1b = W1[:_C], W1[_C:]
    w1d = w1a - w1b
    ones = jnp.ones((_C, 1), jnp.float32)

    grid = (_B, _N // _RB)
    out = pl.pallas_call(
        _edgeconv_body,
        grid=grid,
        in_specs=[
            pl.BlockSpec((1, _N, _C), lambda b, r: (b, 0, 0)),
            pl.BlockSpec((1, _RB, _C), lambda b, r: (b, r, 0)),
            pl.BlockSpec((_C, 1), lambda b, r: (0, 0)),
            pl.BlockSpec((_C, _OUT), lambda b, r: (0, 0)),
            pl.BlockSpec((_C, _OUT), lambda b, r: (0, 0)),
            pl.BlockSpec((_OUT,), lambda b, r: (0,)),
            pl.BlockSpec((_OUT, _OUT), lambda b, r: (0, 0)),
            pl.BlockSpec((_OUT,), lambda b, r: (0,)),
        ],
        out_specs=pl.BlockSpec((1, _RB, _OUT), lambda b, r: (b, r, 0)),
        out_shape=jax.ShapeDtypeStruct((_B, _N, _OUT), jnp.float32),
        scratch_shapes=[pltpu.VMEM((_N, _RB), jnp.float32)],
        interpret=interpret,
    )(xf, xf, ones, w1d, w1b, b1, W2, b2)
    return jnp.transpose(out, (0, 2, 1))[..., None]


# speedup vs baseline: 1.0984x; 1.0984x over previous
import functools

import jax
import jax.numpy as jnp
from jax import lax
from jax.experimental import pallas as pl
from jax.experimental.pallas import tpu as pltpu

_B, _C, _N, _K, _OUT = 16, 64, 1024, 20, 64
_RB = 256

_NATIVE = (((1,), (0,)), ((), ()))


def _edgeconv_body(xb_ref, xbt_ref, xrt_ref, ones_ref, w1dt_ref, w1bt_ref,
                   b1_ref, w2t_ref, b2_ref, out_ref, cur_ref):
    xb = xb_ref[0]
    xrt = xrt_ref[0]
    ones = ones_ref[...]
    sqb = lax.dot_general(xb * xb, ones, _NATIVE,
                          preferred_element_type=jnp.float32)
    sqr = lax.dot_general(ones.T, xrt * xrt, _NATIVE,
                          preferred_element_type=jnp.float32)
    dot = lax.dot_general(xb, xrt, _NATIVE,
                          preferred_element_type=jnp.float32)
    cur_ref[...] = (sqr - 2.0 * dot) + sqb

    vt = lax.dot_general(w1bt_ref[...], xbt_ref[0], _NATIVE,
                         preferred_element_type=jnp.float32)
    ut = lax.dot_general(w1dt_ref[...], xrt, _NATIVE,
                         preferred_element_type=jnp.float32)
    ubt = ut + b1_ref[...]
    w2t = w2t_ref[...]
    b2 = b2_ref[...]

    iota = lax.broadcasted_iota(jnp.int32, (_N, _RB), 0)

    def body(_, acc):
        cur = cur_ref[...]
        m = jnp.min(cur, axis=0, keepdims=True)
        ismin = cur == m
        amin = jnp.min(jnp.where(ismin, iota, _N), axis=0, keepdims=True)
        onehot = iota == amin
        vjt = lax.dot_general(vt, jnp.where(onehot, 1.0, 0.0), _NATIVE,
                              preferred_element_type=jnp.float32)
        e = jnp.maximum(ubt + vjt, 0.0)
        o = jnp.maximum(
            lax.dot_general(w2t, e, _NATIVE,
                            preferred_element_type=jnp.float32) + b2, 0.0)
        cur_ref[...] = jnp.where(onehot, jnp.inf, cur)
        return jnp.maximum(acc, o)

    acc = lax.fori_loop(0, _K, body,
                        jnp.full((_OUT, _RB), -jnp.inf, jnp.float32))
    out_ref[0] = acc


@functools.partial(jax.jit, static_argnames=("interpret",))
def kernel(x, W1, b1, W2, b2, interpret=False):
    xt = x[..., 0]
    xf = jnp.transpose(xt, (0, 2, 1))
    w1a, w1b = W1[:_C], W1[_C:]
    w1dt = (w1a - w1b).T
    w1bt = w1b.T
    ones = jnp.ones((_C, 1), jnp.float32)

    grid = (_B, _N // _RB)
    out = pl.pallas_call(
        _edgeconv_body,
        grid=grid,
        in_specs=[
            pl.BlockSpec((1, _N, _C), lambda b, r: (b, 0, 0)),
            pl.BlockSpec((1, _C, _N), lambda b, r: (b, 0, 0)),
            pl.BlockSpec((1, _C, _RB), lambda b, r: (b, 0, r)),
            pl.BlockSpec((_C, 1), lambda b, r: (0, 0)),
            pl.BlockSpec((_OUT, _C), lambda b, r: (0, 0)),
            pl.BlockSpec((_OUT, _C), lambda b, r: (0, 0)),
            pl.BlockSpec((_OUT, 1), lambda b, r: (0, 0)),
            pl.BlockSpec((_OUT, _OUT), lambda b, r: (0, 0)),
            pl.BlockSpec((_OUT, 1), lambda b, r: (0, 0)),
        ],
        out_specs=pl.BlockSpec((1, _OUT, _RB), lambda b, r: (b, 0, r)),
        out_shape=jax.ShapeDtypeStruct((_B, _OUT, _N), jnp.float32),
        scratch_shapes=[pltpu.VMEM((_N, _RB), jnp.float32)],
        interpret=interpret,
    )(xf, xt, xt, ones, w1dt, w1bt, b1[:, None], W2.T, b2[:, None])
    return out[..., None]


# speedup vs baseline: 1.2314x; 1.1212x over previous
import functools

import jax
import jax.numpy as jnp
from jax import lax
from jax.experimental import pallas as pl
from jax.experimental.pallas import tpu as pltpu

_B, _C, _N, _K, _OUT = 16, 64, 1024, 20, 64
_RB = 256

_NATIVE = (((1,), (0,)), ((), ()))
_MINOR2 = (((1,), (1,)), ((), ()))


def _edgeconv_body(xb_ref, xbt_ref, xr_ref, xrt_ref, w1dt_ref, w1bt_ref,
                   b1_ref, w2t_ref, b2_ref, out_ref, cur_ref):
    xb = xb_ref[0]
    xr = xr_ref[0]
    sqb = jnp.sum(xb * xb, axis=1)
    sqr = jnp.sum(xr * xr, axis=1)
    dot = lax.dot_general(xr, xb, _MINOR2,
                          preferred_element_type=jnp.float32)
    d = sqr[:, None] - 2.0 * dot + sqb[None, :]
    cur_ref[...] = d.T

    vt = lax.dot_general(w1bt_ref[...], xbt_ref[0], _NATIVE,
                         preferred_element_type=jnp.float32)
    ut = lax.dot_general(w1dt_ref[...], xrt_ref[0], _NATIVE,
                         preferred_element_type=jnp.float32)
    ubt = ut + b1_ref[...]
    w2t = w2t_ref[...]
    b2 = b2_ref[...]

    iota = lax.broadcasted_iota(jnp.int32, (_N, _RB), 0)
    neg = jnp.full((_OUT, _RB), -jnp.inf, jnp.float32)

    def body(t, carry):
        acc, vjp = carry
        cur = cur_ref[...]
        m = jnp.min(cur, axis=0, keepdims=True)
        ismin = cur == m
        amin = jnp.min(jnp.where(ismin, iota, _N), axis=0, keepdims=True)
        onehot = iota == amin
        cur_ref[...] = jnp.where(onehot, jnp.inf, cur)
        vjt = lax.dot_general(vt, jnp.where(onehot, 1.0, 0.0), _NATIVE,
                              preferred_element_type=jnp.float32)
        e = jnp.maximum(ubt + vjp, 0.0)
        o = jnp.maximum(
            lax.dot_general(w2t, e, _NATIVE,
                            preferred_element_type=jnp.float32) + b2, 0.0)
        acc = jnp.maximum(acc, jnp.where(t == 0, -jnp.inf, o))
        return acc, vjt

    acc, vjt = lax.fori_loop(0, _K, body, (neg, neg))
    e = jnp.maximum(ubt + vjt, 0.0)
    o = jnp.maximum(
        lax.dot_general(w2t, e, _NATIVE,
                        preferred_element_type=jnp.float32) + b2, 0.0)
    out_ref[0] = jnp.maximum(acc, o)


@functools.partial(jax.jit, static_argnames=("interpret",))
def kernel(x, W1, b1, W2, b2, interpret=False):
    xt = x[..., 0]
    xf = jnp.transpose(xt, (0, 2, 1))
    w1a, w1b = W1[:_C], W1[_C:]
    w1dt = (w1a - w1b).T
    w1bt = w1b.T

    grid = (_B, _N // _RB)
    out = pl.pallas_call(
        _edgeconv_body,
        grid=grid,
        in_specs=[
            pl.BlockSpec((1, _N, _C), lambda b, r: (b, 0, 0)),
            pl.BlockSpec((1, _C, _N), lambda b, r: (b, 0, 0)),
            pl.BlockSpec((1, _RB, _C), lambda b, r: (b, r, 0)),
            pl.BlockSpec((1, _C, _RB), lambda b, r: (b, 0, r)),
            pl.BlockSpec((_OUT, _C), lambda b, r: (0, 0)),
            pl.BlockSpec((_OUT, _C), lambda b, r: (0, 0)),
            pl.BlockSpec((_OUT, 1), lambda b, r: (0, 0)),
            pl.BlockSpec((_OUT, _OUT), lambda b, r: (0, 0)),
            pl.BlockSpec((_OUT, 1), lambda b, r: (0, 0)),
        ],
        out_specs=pl.BlockSpec((1, _OUT, _RB), lambda b, r: (b, 0, r)),
        out_shape=jax.ShapeDtypeStruct((_B, _OUT, _N), jnp.float32),
        scratch_shapes=[pltpu.VMEM((_N, _RB), jnp.float32)],
        interpret=interpret,
    )(xf, xt, xf, xt, w1dt, w1bt, b1[:, None], W2.T, b2[:, None])
    return out[..., None]


# speedup vs baseline: 1.5483x; 1.2573x over previous
import functools

import jax
import jax.numpy as jnp
from jax import lax
from jax.experimental import pallas as pl
from jax.experimental.pallas import tpu as pltpu

_B, _C, _N, _K, _OUT = 16, 64, 1024, 20, 64
_RB = 256

_NATIVE = (((1,), (0,)), ((), ()))


def _edgeconv_body(xb_ref, xbt_ref, xr_ref, xrt_ref, w1dt_ref, w1bt_ref,
                   b1_ref, w2t_ref, b2_ref, out_ref, cur_ref):
    xb = xb_ref[0]
    xr = xr_ref[0]
    sqb = jnp.sum(xb * xb, axis=1, keepdims=True)
    sqr = jnp.sum(xr * xr, axis=1, keepdims=True).T
    dott = lax.dot_general(xb, xrt_ref[0], _NATIVE,
                           preferred_element_type=jnp.float32)
    cur_ref[...] = (sqr - 2.0 * dott) + sqb

    vt = lax.dot_general(w1bt_ref[...], xbt_ref[0], _NATIVE,
                         preferred_element_type=jnp.float32)
    ut = lax.dot_general(w1dt_ref[...], xrt_ref[0], _NATIVE,
                         preferred_element_type=jnp.float32)
    ubt = ut + b1_ref[...]
    w2t = w2t_ref[...]
    b2 = b2_ref[...]

    iota = lax.broadcasted_iota(jnp.int32, (_N, _RB), 0)
    neg = jnp.full((_OUT, _RB), -jnp.inf, jnp.float32)

    def scan_once():
        cur = cur_ref[...]
        m = jnp.min(cur, axis=0, keepdims=True)
        ismin = cur == m
        amin = jnp.min(jnp.where(ismin, iota, _N), axis=0, keepdims=True)
        onehot = iota == amin
        cur_ref[...] = jnp.where(onehot, jnp.inf, cur)
        return lax.dot_general(vt, jnp.where(onehot, 1.0, 0.0), _NATIVE,
                               preferred_element_type=jnp.float32)

    def mlp(vjt):
        e = jnp.maximum(ubt + vjt, 0.0)
        return jnp.maximum(
            lax.dot_general(w2t, e, _NATIVE,
                            preferred_element_type=jnp.float32) + b2, 0.0)

    def body(i, carry):
        acc, vjp = carry
        vja = scan_once()
        vjb = scan_once()
        acc = jnp.maximum(acc, jnp.where(i == 0, -jnp.inf, mlp(vjp)))
        acc = jnp.maximum(acc, mlp(vja))
        return acc, vjb

    acc, vjt = lax.fori_loop(0, _K // 2, body, (neg, neg))
    out_ref[0] = jnp.maximum(acc, mlp(vjt))


@functools.partial(jax.jit, static_argnames=("interpret",))
def kernel(x, W1, b1, W2, b2, interpret=False):
    xt = x[..., 0]
    xf = jnp.transpose(xt, (0, 2, 1))
    w1a, w1b = W1[:_C], W1[_C:]
    w1dt = (w1a - w1b).T
    w1bt = w1b.T

    grid = (_B, _N // _RB)
    out = pl.pallas_call(
        _edgeconv_body,
        grid=grid,
        in_specs=[
            pl.BlockSpec((1, _N, _C), lambda b, r: (b, 0, 0)),
            pl.BlockSpec((1, _C, _N), lambda b, r: (b, 0, 0)),
            pl.BlockSpec((1, _RB, _C), lambda b, r: (b, r, 0)),
            pl.BlockSpec((1, _C, _RB), lambda b, r: (b, 0, r)),
            pl.BlockSpec((_OUT, _C), lambda b, r: (0, 0)),
            pl.BlockSpec((_OUT, _C), lambda b, r: (0, 0)),
            pl.BlockSpec((_OUT, 1), lambda b, r: (0, 0)),
            pl.BlockSpec((_OUT, _OUT), lambda b, r: (0, 0)),
            pl.BlockSpec((_OUT, 1), lambda b, r: (0, 0)),
        ],
        out_specs=pl.BlockSpec((1, _OUT, _RB), lambda b, r: (b, 0, r)),
        out_shape=jax.ShapeDtypeStruct((_B, _OUT, _N), jnp.float32),
        scratch_shapes=[pltpu.VMEM((_N, _RB), jnp.float32)],
        interpret=interpret,
    )(xf, xt, xf, xt, w1dt, w1bt, b1[:, None], W2.T, b2[:, None])
    return out[..., None]


# speedup vs baseline: 1.7341x; 1.1200x over previous
import functools

import jax
import jax.numpy as jnp
from jax import lax
from jax.experimental import pallas as pl
from jax.experimental.pallas import tpu as pltpu

_B, _C, _N, _K, _OUT = 16, 64, 1024, 20, 64
_RB = 256

_NATIVE = (((1,), (0,)), ((), ()))


def _edgeconv_body(xb_ref, xbt_ref, xr_ref, xrt_ref, w1dt_ref, w1bt_ref,
                   b1_ref, w2t_ref, b2_ref, out_ref, cur_ref):
    xb = xb_ref[0]
    xr = xr_ref[0]
    sqb = jnp.sum(xb * xb, axis=1, keepdims=True)
    sqr = jnp.sum(xr * xr, axis=1, keepdims=True).T
    dott = lax.dot_general(xb, xrt_ref[0], _NATIVE,
                           preferred_element_type=jnp.float32)
    cur_ref[...] = (sqr - 2.0 * dott) + sqb

    vt = lax.dot_general(w1bt_ref[...], xbt_ref[0], _NATIVE,
                         preferred_element_type=jnp.float32)
    ut = lax.dot_general(w1dt_ref[...], xrt_ref[0], _NATIVE,
                         preferred_element_type=jnp.float32)
    ubt = ut + b1_ref[...]
    w2t = w2t_ref[...]
    b2 = b2_ref[...]

    iota = lax.broadcasted_iota(jnp.int32, (_N, _RB), 0)
    neg = jnp.full((_OUT, _RB), -jnp.inf, jnp.float32)

    def scan_once():
        cur = cur_ref[...]
        m = jnp.min(cur, axis=0, keepdims=True)
        ismin = cur == m
        amin = jnp.min(jnp.where(ismin, iota, _N), axis=0, keepdims=True)
        onehot = iota == amin
        cur_ref[...] = jnp.where(onehot, jnp.inf, cur)
        return lax.dot_general(vt, jnp.where(onehot, 1.0, 0.0), _NATIVE,
                               preferred_element_type=jnp.float32)

    def mlp(vjt):
        e = jnp.maximum(ubt + vjt, 0.0)
        return jnp.maximum(
            lax.dot_general(w2t, e, _NATIVE,
                            preferred_element_type=jnp.float32) + b2, 0.0)

    def body(i, carry):
        acc, vjp = carry
        vja = scan_once()
        vjb = scan_once()
        vjc = scan_once()
        vjd = scan_once()
        acc = jnp.maximum(acc, jnp.where(i == 0, -jnp.inf, mlp(vjp)))
        acc = jnp.maximum(acc, mlp(vja))
        acc = jnp.maximum(acc, mlp(vjb))
        acc = jnp.maximum(acc, mlp(vjc))
        return acc, vjd

    acc, vjt = lax.fori_loop(0, _K // 4, body, (neg, neg))
    out_ref[0] = jnp.maximum(acc, mlp(vjt))


@functools.partial(jax.jit, static_argnames=("interpret",))
def kernel(x, W1, b1, W2, b2, interpret=False):
    xt = x[..., 0]
    xf = jnp.transpose(xt, (0, 2, 1))
    w1a, w1b = W1[:_C], W1[_C:]
    w1dt = (w1a - w1b).T
    w1bt = w1b.T

    grid = (_B, _N // _RB)
    out = pl.pallas_call(
        _edgeconv_body,
        grid=grid,
        in_specs=[
            pl.BlockSpec((1, _N, _C), lambda b, r: (b, 0, 0)),
            pl.BlockSpec((1, _C, _N), lambda b, r: (b, 0, 0)),
            pl.BlockSpec((1, _RB, _C), lambda b, r: (b, r, 0)),
            pl.BlockSpec((1, _C, _RB), lambda b, r: (b, 0, r)),
            pl.BlockSpec((_OUT, _C), lambda b, r: (0, 0)),
            pl.BlockSpec((_OUT, _C), lambda b, r: (0, 0)),
            pl.BlockSpec((_OUT, 1), lambda b, r: (0, 0)),
            pl.BlockSpec((_OUT, _OUT), lambda b, r: (0, 0)),
            pl.BlockSpec((_OUT, 1), lambda b, r: (0, 0)),
        ],
        out_specs=pl.BlockSpec((1, _OUT, _RB), lambda b, r: (b, 0, r)),
        out_shape=jax.ShapeDtypeStruct((_B, _OUT, _N), jnp.float32),
        scratch_shapes=[pltpu.VMEM((_N, _RB), jnp.float32)],
        interpret=interpret,
    )(xf, xt, xf, xt, w1dt, w1bt, b1[:, None], W2.T, b2[:, None])
    return out[..., None]


# speedup vs baseline: 2.1371x; 1.2324x over previous
import functools

import jax
import jax.numpy as jnp
from jax import lax
from jax.experimental import pallas as pl
from jax.experimental.pallas import tpu as pltpu

_B, _C, _N, _K, _OUT = 16, 64, 1024, 20, 64
_RB = 256

_NATIVE = (((1,), (0,)), ((), ()))


def _edgeconv_body(xb_ref, xbt_ref, xr_ref, xrt_ref, w1dt_ref, w1bt_ref,
                   b1_ref, w2t_ref, b2_ref, out_ref, cur_ref):
    xb = xb_ref[0]
    xr = xr_ref[0]
    sqb = jnp.sum(xb * xb, axis=1, keepdims=True)
    sqr = jnp.sum(xr * xr, axis=1, keepdims=True).T
    dott = lax.dot_general(xb, xrt_ref[0], _NATIVE,
                           preferred_element_type=jnp.float32)
    cur_ref[...] = (sqr - 2.0 * dott) + sqb

    vt = lax.dot_general(w1bt_ref[...], xbt_ref[0], _NATIVE,
                         preferred_element_type=jnp.float32)
    ut = lax.dot_general(w1dt_ref[...], xrt_ref[0], _NATIVE,
                         preferred_element_type=jnp.float32)
    ubt = ut + b1_ref[...]
    w2t = w2t_ref[...]
    b2 = b2_ref[...]

    iota = lax.broadcasted_iota(jnp.int32, (_N, _RB), 0)
    neg = jnp.full((_OUT, _RB), -jnp.inf, jnp.float32)

    def scan_once():
        cur = cur_ref[...]
        m = jnp.min(cur, axis=0, keepdims=True)
        ismin = cur == m
        amin = jnp.min(jnp.where(ismin, iota, _N), axis=0, keepdims=True)
        onehot = iota == amin
        cur_ref[...] = jnp.where(onehot, jnp.inf, cur)
        return lax.dot_general(vt, jnp.where(onehot, 1.0, 0.0), _NATIVE,
                               preferred_element_type=jnp.float32)

    def mlp(vjt):
        e = jnp.maximum(ubt + vjt, 0.0)
        return jnp.maximum(
            lax.dot_general(w2t, e, _NATIVE,
                            preferred_element_type=jnp.float32) + b2, 0.0)

    acc = neg
    for _ in range(_K):
        acc = jnp.maximum(acc, mlp(scan_once()))
    out_ref[0] = acc


@functools.partial(jax.jit, static_argnames=("interpret",))
def kernel(x, W1, b1, W2, b2, interpret=False):
    xt = x[..., 0]
    xf = jnp.transpose(xt, (0, 2, 1))
    w1a, w1b = W1[:_C], W1[_C:]
    w1dt = (w1a - w1b).T
    w1bt = w1b.T

    grid = (_B, _N // _RB)
    out = pl.pallas_call(
        _edgeconv_body,
        grid=grid,
        in_specs=[
            pl.BlockSpec((1, _N, _C), lambda b, r: (b, 0, 0)),
            pl.BlockSpec((1, _C, _N), lambda b, r: (b, 0, 0)),
            pl.BlockSpec((1, _RB, _C), lambda b, r: (b, r, 0)),
            pl.BlockSpec((1, _C, _RB), lambda b, r: (b, 0, r)),
            pl.BlockSpec((_OUT, _C), lambda b, r: (0, 0)),
            pl.BlockSpec((_OUT, _C), lambda b, r: (0, 0)),
            pl.BlockSpec((_OUT, 1), lambda b, r: (0, 0)),
            pl.BlockSpec((_OUT, _OUT), lambda b, r: (0, 0)),
            pl.BlockSpec((_OUT, 1), lambda b, r: (0, 0)),
        ],
        out_specs=pl.BlockSpec((1, _OUT, _RB), lambda b, r: (b, 0, r)),
        out_shape=jax.ShapeDtypeStruct((_B, _OUT, _N), jnp.float32),
        scratch_shapes=[pltpu.VMEM((_N, _RB), jnp.float32)],
        interpret=interpret,
    )(xf, xt, xf, xt, w1dt, w1bt, b1[:, None], W2.T, b2[:, None])
    return out[..., None]


# speedup vs baseline: 2.2420x; 1.0491x over previous
import functools

import jax
import jax.numpy as jnp
from jax import lax
from jax.experimental import pallas as pl
from jax.experimental.pallas import tpu as pltpu

_B, _C, _N, _K, _OUT = 16, 64, 1024, 20, 64
_RB = 256

_NATIVE = (((1,), (0,)), ((), ()))


def _edgeconv_body(xb_ref, xbt_ref, xr_ref, xrt_ref, w1dt_ref, w1bt_ref,
                   b1_ref, w2t_ref, b2_ref, out_ref, cur_ref):
    xb = xb_ref[0]
    xr = xr_ref[0]
    sqb = jnp.sum(xb * xb, axis=1, keepdims=True)
    sqr = jnp.sum(xr * xr, axis=1, keepdims=True).T
    dott = lax.dot_general(xb, xrt_ref[0], _NATIVE,
                           preferred_element_type=jnp.float32)
    cur_ref[...] = (sqr - 2.0 * dott) + sqb

    vt = lax.dot_general(w1bt_ref[...], xbt_ref[0], _NATIVE,
                         preferred_element_type=jnp.float32)
    ut = lax.dot_general(w1dt_ref[...], xrt_ref[0], _NATIVE,
                         preferred_element_type=jnp.float32)
    ubt = ut + b1_ref[...]
    w2t = w2t_ref[...]
    b2 = b2_ref[...]

    iota = lax.broadcasted_iota(jnp.int32, (_N, _RB), 0)
    neg = jnp.full((_OUT, _RB), -jnp.inf, jnp.float32)

    def scan_once():
        cur = cur_ref[...]
        v, ix = cur, iota
        n = _N
        while n > 1:
            h = n // 2
            v1, v2 = v[:h], v[h:]
            take = v2 < v1
            v = jnp.where(take, v2, v1)
            ix = jnp.where(take, ix[h:], ix[:h])
            n = h
        amin = ix
        onehot = iota == amin
        cur_ref[...] = jnp.where(onehot, jnp.inf, cur)
        return lax.dot_general(vt, jnp.where(onehot, 1.0, 0.0), _NATIVE,
                               preferred_element_type=jnp.float32)

    def mlp(vjt):
        e = jnp.maximum(ubt + vjt, 0.0)
        return jnp.maximum(
            lax.dot_general(w2t, e, _NATIVE,
                            preferred_element_type=jnp.float32) + b2, 0.0)

    acc = neg
    for _ in range(_K):
        acc = jnp.maximum(acc, mlp(scan_once()))
    out_ref[0] = acc


@functools.partial(jax.jit, static_argnames=("interpret",))
def kernel(x, W1, b1, W2, b2, interpret=False):
    xt = x[..., 0]
    xf = jnp.transpose(xt, (0, 2, 1))
    w1a, w1b = W1[:_C], W1[_C:]
    w1dt = (w1a - w1b).T
    w1bt = w1b.T

    grid = (_B, _N // _RB)
    out = pl.pallas_call(
        _edgeconv_body,
        grid=grid,
        in_specs=[
            pl.BlockSpec((1, _N, _C), lambda b, r: (b, 0, 0)),
            pl.BlockSpec((1, _C, _N), lambda b, r: (b, 0, 0)),
            pl.BlockSpec((1, _RB, _C), lambda b, r: (b, r, 0)),
            pl.BlockSpec((1, _C, _RB), lambda b, r: (b, 0, r)),
            pl.BlockSpec((_OUT, _C), lambda b, r: (0, 0)),
            pl.BlockSpec((_OUT, _C), lambda b, r: (0, 0)),
            pl.BlockSpec((_OUT, 1), lambda b, r: (0, 0)),
            pl.BlockSpec((_OUT, _OUT), lambda b, r: (0, 0)),
            pl.BlockSpec((_OUT, 1), lambda b, r: (0, 0)),
        ],
        out_specs=pl.BlockSpec((1, _OUT, _RB), lambda b, r: (b, 0, r)),
        out_shape=jax.ShapeDtypeStruct((_B, _OUT, _N), jnp.float32),
        scratch_shapes=[pltpu.VMEM((_N, _RB), jnp.float32)],
        interpret=interpret,
    )(xf, xt, xf, xt, w1dt, w1bt, b1[:, None], W2.T, b2[:, None])
    return out[..., None]


# speedup vs baseline: 2.6609x; 1.1868x over previous
import functools

import jax
import jax.numpy as jnp
from jax import lax
from jax.experimental import pallas as pl
from jax.experimental.pallas import tpu as pltpu

_B, _C, _N, _K, _OUT = 16, 64, 1024, 20, 64
_RB = 512

_NATIVE = (((1,), (0,)), ((), ()))


def _edgeconv_body(xb_ref, xbt_ref, xr_ref, xrt_ref, w1dt_ref, w1bt_ref,
                   b1_ref, w2t_ref, b2_ref, out_ref, cur_ref):
    xb = xb_ref[0]
    xr = xr_ref[0]
    sqb = jnp.sum(xb * xb, axis=1, keepdims=True)
    sqr = jnp.sum(xr * xr, axis=1, keepdims=True).T
    dott = lax.dot_general(xb, xrt_ref[0], _NATIVE,
                           preferred_element_type=jnp.float32)
    cur_ref[...] = (sqr - 2.0 * dott) + sqb

    vt = lax.dot_general(w1bt_ref[...], xbt_ref[0], _NATIVE,
                         preferred_element_type=jnp.float32)
    ut = lax.dot_general(w1dt_ref[...], xrt_ref[0], _NATIVE,
                         preferred_element_type=jnp.float32)
    ubt = ut + b1_ref[...]
    w2t = w2t_ref[...]
    b2 = b2_ref[...]

    iota = lax.broadcasted_iota(jnp.int32, (_N, _RB), 0)
    neg = jnp.full((_OUT, _RB), -jnp.inf, jnp.float32)

    def scan_once():
        cur = cur_ref[...]
        v, ix = cur, iota
        n = _N
        while n > 1:
            h = n // 2
            v1, v2 = v[:h], v[h:]
            take = v2 < v1
            v = jnp.where(take, v2, v1)
            ix = jnp.where(take, ix[h:], ix[:h])
            n = h
        amin = ix
        onehot = iota == amin
        cur_ref[...] = jnp.where(onehot, jnp.inf, cur)
        return lax.dot_general(vt, jnp.where(onehot, 1.0, 0.0), _NATIVE,
                               preferred_element_type=jnp.float32)

    def mlp(vjt):
        e = jnp.maximum(ubt + vjt, 0.0)
        return jnp.maximum(
            lax.dot_general(w2t, e, _NATIVE,
                            preferred_element_type=jnp.float32) + b2, 0.0)

    acc = neg
    for _ in range(_K):
        acc = jnp.maximum(acc, mlp(scan_once()))
    out_ref[0] = acc


@functools.partial(jax.jit, static_argnames=("interpret",))
def kernel(x, W1, b1, W2, b2, interpret=False):
    xt = x[..., 0]
    xf = jnp.transpose(xt, (0, 2, 1))
    w1a, w1b = W1[:_C], W1[_C:]
    w1dt = (w1a - w1b).T
    w1bt = w1b.T

    grid = (_B, _N // _RB)
    out = pl.pallas_call(
        _edgeconv_body,
        grid=grid,
        in_specs=[
            pl.BlockSpec((1, _N, _C), lambda b, r: (b, 0, 0)),
            pl.BlockSpec((1, _C, _N), lambda b, r: (b, 0, 0)),
            pl.BlockSpec((1, _RB, _C), lambda b, r: (b, r, 0)),
            pl.BlockSpec((1, _C, _RB), lambda b, r: (b, 0, r)),
            pl.BlockSpec((_OUT, _C), lambda b, r: (0, 0)),
            pl.BlockSpec((_OUT, _C), lambda b, r: (0, 0)),
            pl.BlockSpec((_OUT, 1), lambda b, r: (0, 0)),
            pl.BlockSpec((_OUT, _OUT), lambda b, r: (0, 0)),
            pl.BlockSpec((_OUT, 1), lambda b, r: (0, 0)),
        ],
        out_specs=pl.BlockSpec((1, _OUT, _RB), lambda b, r: (b, 0, r)),
        out_shape=jax.ShapeDtypeStruct((_B, _OUT, _N), jnp.float32),
        scratch_shapes=[pltpu.VMEM((_N, _RB), jnp.float32)],
        interpret=interpret,
    )(xf, xt, xf, xt, w1dt, w1bt, b1[:, None], W2.T, b2[:, None])
    return out[..., None]


# speedup vs baseline: 2.7338x; 1.0274x over previous
import functools

import jax
import jax.numpy as jnp
from jax import lax
from jax.experimental import pallas as pl
from jax.experimental.pallas import tpu as pltpu

_B, _C, _N, _K, _OUT = 16, 64, 1024, 20, 64
_RB = 1024

_NATIVE = (((1,), (0,)), ((), ()))


def _edgeconv_body(xb_ref, xbt_ref, xr_ref, xrt_ref, w1dt_ref, w1bt_ref,
                   b1_ref, w2t_ref, b2_ref, out_ref, cur_ref):
    xb = xb_ref[0]
    xr = xr_ref[0]
    sqb = jnp.sum(xb * xb, axis=1, keepdims=True)
    sqr = jnp.sum(xr * xr, axis=1, keepdims=True).T
    dott = lax.dot_general(xb, xrt_ref[0], _NATIVE,
                           preferred_element_type=jnp.float32)
    cur_ref[...] = (sqr - 2.0 * dott) + sqb

    vt = lax.dot_general(w1bt_ref[...], xbt_ref[0], _NATIVE,
                         preferred_element_type=jnp.float32)
    ut = lax.dot_general(w1dt_ref[...], xrt_ref[0], _NATIVE,
                         preferred_element_type=jnp.float32)
    ubt = ut + b1_ref[...]
    w2t = w2t_ref[...]
    b2 = b2_ref[...]

    iota = lax.broadcasted_iota(jnp.int32, (_N, _RB), 0)
    neg = jnp.full((_OUT, _RB), -jnp.inf, jnp.float32)

    def scan_once():
        cur = cur_ref[...]
        v, ix = cur, iota
        n = _N
        while n > 1:
            h = n // 2
            v1, v2 = v[:h], v[h:]
            take = v2 < v1
            v = jnp.where(take, v2, v1)
            ix = jnp.where(take, ix[h:], ix[:h])
            n = h
        amin = ix
        onehot = iota == amin
        cur_ref[...] = jnp.where(onehot, jnp.inf, cur)
        return lax.dot_general(vt, jnp.where(onehot, 1.0, 0.0), _NATIVE,
                               preferred_element_type=jnp.float32)

    def mlp(vjt):
        e = jnp.maximum(ubt + vjt, 0.0)
        return jnp.maximum(
            lax.dot_general(w2t, e, _NATIVE,
                            preferred_element_type=jnp.float32) + b2, 0.0)

    acc = neg
    for _ in range(_K):
        acc = jnp.maximum(acc, mlp(scan_once()))
    out_ref[0] = acc


@functools.partial(jax.jit, static_argnames=("interpret",))
def kernel(x, W1, b1, W2, b2, interpret=False):
    xt = x[..., 0]
    xf = jnp.transpose(xt, (0, 2, 1))
    w1a, w1b = W1[:_C], W1[_C:]
    w1dt = (w1a - w1b).T
    w1bt = w1b.T

    grid = (_B, _N // _RB)
    out = pl.pallas_call(
        _edgeconv_body,
        grid=grid,
        in_specs=[
            pl.BlockSpec((1, _N, _C), lambda b, r: (b, 0, 0)),
            pl.BlockSpec((1, _C, _N), lambda b, r: (b, 0, 0)),
            pl.BlockSpec((1, _RB, _C), lambda b, r: (b, r, 0)),
            pl.BlockSpec((1, _C, _RB), lambda b, r: (b, 0, r)),
            pl.BlockSpec((_OUT, _C), lambda b, r: (0, 0)),
            pl.BlockSpec((_OUT, _C), lambda b, r: (0, 0)),
            pl.BlockSpec((_OUT, 1), lambda b, r: (0, 0)),
            pl.BlockSpec((_OUT, _OUT), lambda b, r: (0, 0)),
            pl.BlockSpec((_OUT, 1), lambda b, r: (0, 0)),
        ],
        out_specs=pl.BlockSpec((1, _OUT, _RB), lambda b, r: (b, 0, r)),
        out_shape=jax.ShapeDtypeStruct((_B, _OUT, _N), jnp.float32),
        scratch_shapes=[pltpu.VMEM((_N, _RB), jnp.float32)],
        interpret=interpret,
    )(xf, xt, xf, xt, w1dt, w1bt, b1[:, None], W2.T, b2[:, None])
    return out[..., None]
